# baseline (speedup 1.0000x reference)
import jax
import jax.numpy as jnp
from jax import lax
from jax.experimental import pallas as pl
from jax.experimental.pallas import tpu as pltpu
from jax.experimental.pallas import tpu_sc as plsc

H = 64
W = 64
NW = 32
CHUNK = 4
NBUF = 6
LAG = 3
LANES = 16
ROW = H * W


def _sc_kernel(fov_hbm, act_hbm, pos_hbm, tab_hbm, val_hbm,
               out_hbm, pos_out_hbm, mask_out_hbm,
               bufs, abuf, pbuf, pobuf, mobuf, tabbuf, valbuf,
               gidxbuf, cellbuf, f2buf,
               sis, sos, sm, sg):
    B = fov_hbm.shape[0] // ROW
    rpw = B // NW
    nch = rpw // CHUNK
    wid = lax.axis_index("s") * 2 + lax.axis_index("c")
    base = wid * rpw

    def start_in(g, b):
        pltpu.make_async_copy(
            fov_hbm.at[pl.ds((base + g * CHUNK) * ROW, CHUNK * ROW)],
            bufs[b], sis[b]).start()

    def wait_in(g, b):
        pltpu.make_async_copy(
            fov_hbm.at[pl.ds((base + g * CHUNK) * ROW, CHUNK * ROW)],
            bufs[b], sis[b]).wait()

    def start_out(g, b):
        pltpu.make_async_copy(
            bufs[b], out_hbm.at[pl.ds((base + g * CHUNK) * ROW, CHUNK * ROW)],
            sos[b]).start()

    def wait_out(g, b):
        pltpu.make_async_copy(
            bufs[b], out_hbm.at[pl.ds((base + g * CHUNK) * ROW, CHUNK * ROW)],
            sos[b]).wait()

    pltpu.make_async_copy(act_hbm.at[pl.ds(base, rpw)],
                          abuf.at[pl.ds(0, rpw)], sm).start()
    pltpu.make_async_copy(pos_hbm.at[pl.ds(2 * base, 2 * rpw)],
                          pbuf.at[pl.ds(0, 2 * rpw)], sm).start()
    pltpu.make_async_copy(tab_hbm, tabbuf, sm).start()
    pltpu.make_async_copy(val_hbm, valbuf, sm).start()
    for b in range(NBUF):
        start_in(b, b)
    pltpu.make_async_copy(act_hbm.at[pl.ds(base, rpw)],
                          abuf.at[pl.ds(0, rpw)], sm).wait()
    pltpu.make_async_copy(pos_hbm.at[pl.ds(2 * base, 2 * rpw)],
                          pbuf.at[pl.ds(0, 2 * rpw)], sm).wait()
    pltpu.make_async_copy(tab_hbm, tabbuf, sm).wait()
    pltpu.make_async_copy(val_hbm, valbuf, sm).wait()

    k16 = lax.iota(jnp.int32, LANES)
    lane_ok = k16 < CHUNK
    val_vec = valbuf[...]

    def pre(j, carry):
        rows = j * LANES + k16
        aidx = jnp.clip(abuf[pl.ds(j * LANES, LANES)], 0, 8)
        ys = plsc.load_gather(pbuf, [2 * rows])
        xs = plsc.load_gather(pbuf, [2 * rows + 1])
        dy = plsc.load_gather(tabbuf, [2 * aidx])
        dx = plsc.load_gather(tabbuf, [2 * aidx + 1])
        ny = jnp.clip(ys + dy, 0, H - 1)
        nx = jnp.clip(xs + dx, 0, W - 1)
        gidxbuf[pl.ds(j * LANES, LANES)] = (base + rows) * ROW + ny * W + nx
        return carry

    lax.fori_loop(0, rpw // LANES, pre, 0)

    for q in range(rpw // 128):
        pltpu.make_async_copy(
            fov_hbm.at[gidxbuf.at[pl.ds(q * 128, 128)]],
            cellbuf.at[pl.ds(q * 128, 128)], sg).start()
    for q in range(rpw // 128):
        pltpu.make_async_copy(
            fov_hbm.at[gidxbuf.at[pl.ds(q * 128, 128)]],
            cellbuf.at[pl.ds(q * 128, 128)], sg).wait()

    def mid(j, carry):
        rows = j * LANES + k16
        aidx = jnp.clip(abuf[pl.ds(j * LANES, LANES)], 0, 8)
        ys = plsc.load_gather(pbuf, [2 * rows])
        xs = plsc.load_gather(pbuf, [2 * rows + 1])
        dy = plsc.load_gather(tabbuf, [2 * aidx])
        dx = plsc.load_gather(tabbuf, [2 * aidx + 1])
        cell = cellbuf[pl.ds(j * LANES, LANES)]
        blocked = cell == 1.0
        dy2 = jnp.where(blocked, 0, dy)
        dx2 = jnp.where(blocked, 0, dx)
        y2 = ys + dy2
        x2 = xs + dx2
        lrow = rows - (rows // CHUNK) * CHUNK
        f2buf[pl.ds(j * LANES, LANES)] = (
            lrow * ROW + jnp.clip(y2, 0, H - 1) * W + jnp.clip(x2, 0, W - 1))
        plsc.store_scatter(pobuf, [2 * rows], y2)
        plsc.store_scatter(pobuf, [2 * rows + 1], x2)
        mobuf[pl.ds(j * LANES, LANES)] = jnp.where(cell == 2.0, 1, 0)
        return carry

    lax.fori_loop(0, rpw // LANES, mid, 0)

    pltpu.make_async_copy(pobuf.at[pl.ds(0, 2 * rpw)],
                          pos_out_hbm.at[pl.ds(2 * base, 2 * rpw)], sm).start()
    pltpu.make_async_copy(mobuf.at[pl.ds(0, rpw)],
                          mask_out_hbm.at[pl.ds(base, rpw)], sm).start()

    def process(g, b):
        f2 = jnp.where(lane_ok, f2buf[pl.ds(g * CHUNK, LANES)], 0)
        plsc.store_scatter(bufs[b], [f2], val_vec, mask=lane_ok)

    def turn(g, b):
        wait_in(g, b)
        process(g, b)
        start_out(g, b)

        gm = jnp.maximum(g - LAG, 0)
        bm = (b + NBUF - LAG) % NBUF

        @pl.when(jnp.logical_and(g >= LAG, g - LAG + NBUF < nch))
        def _():
            wait_out(gm, bm)
            start_in(gm + NBUF, bm)

    def body(i, carry):
        for b in range(NBUF):
            turn(i * NBUF + b, b)
        return carry

    main_iters = nch // NBUF
    lax.fori_loop(0, main_iters, body, 0)
    for g in range(main_iters * NBUF, nch):
        turn(g, g % NBUF)

    pltpu.make_async_copy(pobuf.at[pl.ds(0, 2 * rpw)],
                          pos_out_hbm.at[pl.ds(2 * base, 2 * rpw)], sm).wait()
    pltpu.make_async_copy(mobuf.at[pl.ds(0, rpw)],
                          mask_out_hbm.at[pl.ds(base, rpw)], sm).wait()
    for g in range(nch - 2 * LAG, nch):
        wait_out(g, g % NBUF)


def _sc_run(fov_flat, act_flat, pos_flat, tab_pad, val_arr):
    B = fov_flat.shape[0] // ROW
    rpw = B // NW
    mesh = plsc.VectorSubcoreMesh(core_axis_name="c", subcore_axis_name="s")
    meta_i32 = pltpu.VMEM((rpw + LANES,), jnp.int32)
    f = pl.kernel(
        _sc_kernel,
        out_type=[
            jax.ShapeDtypeStruct((B * ROW,), jnp.float32),
            jax.ShapeDtypeStruct((2 * B,), jnp.int32),
            jax.ShapeDtypeStruct((B,), jnp.int32),
        ],
        mesh=mesh,
        compiler_params=pltpu.CompilerParams(needs_layout_passes=False),
        scratch_types=[
            [pltpu.VMEM((CHUNK * ROW,), jnp.float32)] * NBUF,
            meta_i32,
            pltpu.VMEM((2 * rpw + 2 * LANES,), jnp.int32),
            pltpu.VMEM((2 * rpw + 2 * LANES,), jnp.int32),
            meta_i32,
            pltpu.VMEM((32,), jnp.int32),
            pltpu.VMEM((LANES,), jnp.float32),
            meta_i32,
            pltpu.VMEM((rpw + LANES,), jnp.float32),
            meta_i32,
            [pltpu.SemaphoreType.DMA] * NBUF,
            [pltpu.SemaphoreType.DMA] * NBUF,
            pltpu.SemaphoreType.DMA,
            pltpu.SemaphoreType.DMA,
        ],
    )
    return f(fov_flat, act_flat, pos_flat, tab_pad, val_arr)


def kernel(fov, batch_logit_prob, batch_top_k_prob, batch_action_idx,
           possible_actions, batch_agent_current_pos, step):
    B = fov.shape[0]
    val_arr = jnp.full((LANES,), 3.0 + jnp.asarray(step, jnp.float32),
                       jnp.float32)
    tab_pad = jnp.zeros((32,), jnp.int32).at[:18].set(
        possible_actions.reshape(18))
    new_fov, pos_out, tmask = _sc_run(
        fov.reshape(B * ROW),
        batch_action_idx.reshape(B),
        batch_agent_current_pos.reshape(2 * B),
        tab_pad,
        val_arr)
    return (new_fov.reshape(B, H, W), pos_out.reshape(B, 2),
            tmask.astype(bool),
            batch_action_idx, batch_logit_prob, batch_top_k_prob)


# speedup vs baseline: 2.0507x; 2.0507x over previous
import jax
import jax.numpy as jnp
from jax import lax
from jax.experimental import pallas as pl
from jax.experimental.pallas import tpu as pltpu
from jax.experimental.pallas import tpu_sc as plsc

H = 64
W = 64
NW = 32
CHUNK = 4
NBUF = 6
LAG = 3
LANES = 16


def _sc_kernel(fov_hbm, act_hbm, pos_hbm, tab_hbm, val_hbm,
               out_hbm, pos_out_hbm, mask_out_hbm,
               bufs, abuf, pbuf, pobuf, mobuf, tabbuf, valbuf,
               ysbuf, xsbuf, dybuf, dxbuf, f1buf, f0buf, cellbuf,
               sis, sos, sm):
    B = fov_hbm.shape[0]
    rpw = B // NW
    nch = rpw // CHUNK
    wid = lax.axis_index("s") * 2 + lax.axis_index("c")
    base = wid * rpw

    def start_in(g, b):
        pltpu.make_async_copy(
            fov_hbm.at[pl.ds(base + g * CHUNK, CHUNK)], bufs[b], sis[b]).start()

    def wait_in(g, b):
        pltpu.make_async_copy(
            fov_hbm.at[pl.ds(base + g * CHUNK, CHUNK)], bufs[b], sis[b]).wait()

    def start_out(g, b):
        pltpu.make_async_copy(
            bufs[b], out_hbm.at[pl.ds(base + g * CHUNK, CHUNK)], sos[b]).start()

    def wait_out(g, b):
        pltpu.make_async_copy(
            bufs[b], out_hbm.at[pl.ds(base + g * CHUNK, CHUNK)], sos[b]).wait()

    pltpu.make_async_copy(act_hbm.at[pl.ds(base, rpw)],
                          abuf.at[pl.ds(0, rpw)], sm).start()
    pltpu.make_async_copy(pos_hbm.at[pl.ds(2 * base, 2 * rpw)],
                          pbuf.at[pl.ds(0, 2 * rpw)], sm).start()
    pltpu.make_async_copy(tab_hbm, tabbuf, sm).start()
    pltpu.make_async_copy(val_hbm, valbuf, sm).start()
    for b in range(NBUF):
        start_in(b, b)
    pltpu.make_async_copy(act_hbm.at[pl.ds(base, rpw)],
                          abuf.at[pl.ds(0, rpw)], sm).wait()
    pltpu.make_async_copy(pos_hbm.at[pl.ds(2 * base, 2 * rpw)],
                          pbuf.at[pl.ds(0, 2 * rpw)], sm).wait()
    pltpu.make_async_copy(tab_hbm, tabbuf, sm).wait()
    pltpu.make_async_copy(val_hbm, valbuf, sm).wait()

    k16 = lax.iota(jnp.int32, LANES)
    lane_ok = k16 < CHUNK
    krow = jnp.where(lane_ok, k16, 0)
    val_vec = valbuf[...]

    def pre(j, carry):
        rows = j * LANES + k16
        aidx = jnp.clip(abuf[pl.ds(j * LANES, LANES)], 0, 8)
        ys = plsc.load_gather(pbuf, [2 * rows])
        xs = plsc.load_gather(pbuf, [2 * rows + 1])
        dy = plsc.load_gather(tabbuf, [2 * aidx])
        dx = plsc.load_gather(tabbuf, [2 * aidx + 1])
        ny = jnp.clip(ys + dy, 0, H - 1)
        nx = jnp.clip(xs + dx, 0, W - 1)
        ysbuf[pl.ds(j * LANES, LANES)] = ys
        xsbuf[pl.ds(j * LANES, LANES)] = xs
        dybuf[pl.ds(j * LANES, LANES)] = dy
        dxbuf[pl.ds(j * LANES, LANES)] = dx
        f1buf[pl.ds(j * LANES, LANES)] = ny * W + nx
        f0buf[pl.ds(j * LANES, LANES)] = ys * W + xs
        return carry

    lax.fori_loop(0, rpw // LANES, pre, 0)

    def process(g, b):
        rl = g * CHUNK
        buf = bufs[b]
        f1 = jnp.where(lane_ok, f1buf[pl.ds(rl, LANES)], 0)
        cell = plsc.load_gather(buf, [krow, f1], mask=lane_ok)
        cellbuf[pl.ds(rl, LANES)] = cell
        f2 = jnp.where(cell == 1.0, f0buf[pl.ds(rl, LANES)], f1)
        plsc.store_scatter(buf, [krow, f2], val_vec, mask=lane_ok)

    def turn(g, b):
        wait_in(g, b)
        process(g, b)
        start_out(g, b)

        gm = jnp.maximum(g - LAG, 0)
        bm = (b + NBUF - LAG) % NBUF

        @pl.when(jnp.logical_and(g >= LAG, g - LAG + NBUF < nch))
        def _():
            wait_out(gm, bm)
            start_in(gm + NBUF, bm)

    def body(i, carry):
        for b in range(NBUF):
            turn(i * NBUF + b, b)
        return carry

    main_iters = nch // NBUF
    lax.fori_loop(0, main_iters, body, 0)
    for g in range(main_iters * NBUF, nch):
        turn(g, g % NBUF)

    def post(j, carry):
        rows = j * LANES + k16
        cell = cellbuf[pl.ds(j * LANES, LANES)]
        blocked = cell == 1.0
        y2 = ysbuf[pl.ds(j * LANES, LANES)] + jnp.where(
            blocked, 0, dybuf[pl.ds(j * LANES, LANES)])
        x2 = xsbuf[pl.ds(j * LANES, LANES)] + jnp.where(
            blocked, 0, dxbuf[pl.ds(j * LANES, LANES)])
        plsc.store_scatter(pobuf, [2 * rows], y2)
        plsc.store_scatter(pobuf, [2 * rows + 1], x2)
        mobuf[pl.ds(j * LANES, LANES)] = jnp.where(cell == 2.0, 1, 0)
        return carry

    lax.fori_loop(0, rpw // LANES, post, 0)

    pltpu.make_async_copy(pobuf.at[pl.ds(0, 2 * rpw)],
                          pos_out_hbm.at[pl.ds(2 * base, 2 * rpw)], sm).start()
    pltpu.make_async_copy(pobuf.at[pl.ds(0, 2 * rpw)],
                          pos_out_hbm.at[pl.ds(2 * base, 2 * rpw)], sm).wait()
    pltpu.make_async_copy(mobuf.at[pl.ds(0, rpw)],
                          mask_out_hbm.at[pl.ds(base, rpw)], sm).start()
    pltpu.make_async_copy(mobuf.at[pl.ds(0, rpw)],
                          mask_out_hbm.at[pl.ds(base, rpw)], sm).wait()
    for g in range(nch - 2 * LAG, nch):
        wait_out(g, g % NBUF)


def _sc_run(fov2d, act_flat, pos_flat, tab_pad, val_arr):
    B = fov2d.shape[0]
    rpw = B // NW
    mesh = plsc.VectorSubcoreMesh(core_axis_name="c", subcore_axis_name="s")
    meta_i32 = pltpu.VMEM((rpw + LANES,), jnp.int32)
    f = pl.kernel(
        _sc_kernel,
        out_type=[
            jax.ShapeDtypeStruct((B, H * W), jnp.float32),
            jax.ShapeDtypeStruct((2 * B,), jnp.int32),
            jax.ShapeDtypeStruct((B,), jnp.int32),
        ],
        mesh=mesh,
        compiler_params=pltpu.CompilerParams(needs_layout_passes=False),
        scratch_types=[
            [pltpu.VMEM((CHUNK, H * W), jnp.float32)] * NBUF,
            meta_i32,
            pltpu.VMEM((2 * rpw + 2 * LANES,), jnp.int32),
            pltpu.VMEM((2 * rpw + 2 * LANES,), jnp.int32),
            meta_i32,
            pltpu.VMEM((32,), jnp.int32),
            pltpu.VMEM((LANES,), jnp.float32),
            meta_i32,
            meta_i32,
            meta_i32,
            meta_i32,
            meta_i32,
            meta_i32,
            pltpu.VMEM((rpw + LANES,), jnp.float32),
            [pltpu.SemaphoreType.DMA] * NBUF,
            [pltpu.SemaphoreType.DMA] * NBUF,
            pltpu.SemaphoreType.DMA,
        ],
    )
    return f(fov2d, act_flat, pos_flat, tab_pad, val_arr)


def kernel(fov, batch_logit_prob, batch_top_k_prob, batch_action_idx,
           possible_actions, batch_agent_current_pos, step):
    B = fov.shape[0]
    val_arr = jnp.full((LANES,), 3.0 + jnp.asarray(step, jnp.float32),
                       jnp.float32)
    tab_pad = jnp.zeros((32,), jnp.int32).at[:18].set(
        possible_actions.reshape(18))
    new_fov, pos_out, tmask = _sc_run(
        fov.reshape(B, H * W),
        batch_action_idx.reshape(B),
        batch_agent_current_pos.reshape(2 * B),
        tab_pad,
        val_arr)
    return (new_fov.reshape(B, H, W), pos_out.reshape(B, 2),
            tmask.astype(bool),
            batch_action_idx, batch_logit_prob, batch_top_k_prob)
